# baseline (speedup 1.0000x reference)
import functools

import jax
import jax.numpy as jnp
from jax import lax
from jax.experimental import pallas as pl
from jax.experimental.pallas import tpu as pltpu
from jax.experimental.pallas import tpu_sc as plsc

N_NODES = 10000
N_EDGES = 160000
M = 2 * N_EDGES
A = 128
E_FEAT = 112
ANG = 16
ZDIM = 384

NC = 2
NS = 16
NW = NC * NS
CE = 80
NCHK = N_EDGES // CE
NT = (NCHK + NW - 1) // NW
NPAIR = (NT + 1) // 2


def _sc_assemble(atom_fea, edge_aug, i0, i1, ij):
    mesh = plsc.VectorSubcoreMesh(core_axis_name="c", subcore_axis_name="s")

    @functools.partial(
        pl.kernel,
        out_type=jax.ShapeDtypeStruct((N_EDGES, 6 * A), jnp.float32),
        mesh=mesh,
        scratch_types=[
            pltpu.VMEM((2 * CE,), jnp.int32),
            pltpu.VMEM((2 * CE,), jnp.int32),
            pltpu.VMEM((2 * CE,), jnp.int32),
            pltpu.VMEM((2 * CE,), jnp.int32),
            pltpu.VMEM((2 * CE,), jnp.int32),
            pltpu.VMEM((2 * CE,), jnp.int32),
            pltpu.VMEM((2 * CE, A), jnp.float32),
            pltpu.VMEM((2 * CE, A), jnp.float32),
            pltpu.VMEM((2 * CE, A), jnp.float32),
            pltpu.VMEM((2 * CE, A), jnp.float32),
            pltpu.VMEM((2 * CE, A), jnp.float32),
            pltpu.VMEM((2 * CE, A), jnp.float32),
            pltpu.SemaphoreType.DMA,
            pltpu.SemaphoreType.DMA,
            pltpu.SemaphoreType.DMA,
            pltpu.SemaphoreType.DMA,
            pltpu.SemaphoreType.DMA,
            pltpu.SemaphoreType.DMA,
        ],
    )
    def k(atom_hbm, edge_hbm, i0_hbm, i1_hbm, ij_hbm,
          z_hbm,
          x00, x01, x02, x10, x11, x12,
          b00, b01, b02, b10, b11, b12,
          sg0, sg1, sw0, sw1, si0, si1):
        wid = lax.axis_index("s") * NC + lax.axis_index("c")
        xraw = ((x00, x01, x02), (x10, x11, x12))
        bufs = ((b00, b01, b02), (b10, b11, b12))
        sg = (sg0, sg1)
        sw = (sw0, sw1)
        si = (si0, si1)
        idx_hbms = (i0_hbm, i1_hbm, ij_hbm)
        tabs = (atom_hbm, atom_hbm, edge_hbm)

        def chunk_of(t):
            return wid + t * NW

        def cond(t):
            return chunk_of(t) < NCHK

        def gathers_start(t, s):
            base = 2 * chunk_of(t) * CE
            cps = [pltpu.async_copy(idx_hbms[g].at[pl.ds(base, 2 * CE)],
                                    xraw[s][g], si[s]) for g in range(3)]
            for cp in cps:
                cp.wait()
            for g in range(3):
                pltpu.async_copy(tabs[g].at[xraw[s][g]], bufs[s][g], sg[s])

        def gathers_wait(s):
            for g in range(3):
                pltpu.make_async_copy(tabs[g].at[xraw[s][g]],
                                      bufs[s][g], sg[s]).wait()

        def writes_start(t, s):
            rows = pl.ds(chunk_of(t) * CE, CE)
            for g in range(3):
                pltpu.async_copy(bufs[s][g].reshape(CE, 2 * A),
                                 z_hbm.at[rows, pl.ds(g * 2 * A, 2 * A)],
                                 sw[s])

        def writes_wait(s):
            rows = pl.ds(0, CE)
            for g in range(3):
                pltpu.make_async_copy(bufs[s][g].reshape(CE, 2 * A),
                                      z_hbm.at[rows, pl.ds(g * 2 * A, 2 * A)],
                                      sw[s]).wait()

        gathers_start(0, 0)

        def pair(tt, carry):
            t0 = 2 * tt
            t1 = t0 + 1
            t2 = t0 + 2

            @pl.when(jnp.logical_and(cond(t1), t1 >= 3))
            def _():
                writes_wait(1)

            @pl.when(cond(t1))
            def _():
                gathers_start(t1, 1)

            @pl.when(cond(t0))
            def _():
                gathers_wait(0)
                writes_start(t0, 0)

            @pl.when(cond(t2))
            def _():
                writes_wait(0)
                gathers_start(t2, 0)

            @pl.when(cond(t1))
            def _():
                gathers_wait(1)
                writes_start(t1, 1)

            return carry

        lax.fori_loop(0, NPAIR, pair, 0)
        writes_wait(0)
        writes_wait(1)

    return k(atom_fea, edge_aug, i0, i1, ij)


BE = 3200
DCOL = 2 * A + E_FEAT


def _tc_body(zp_ref, ang_ref, ef_ref, wfs_ref, wang_ref, bfs_ref,
             we1_ref, be1_ref, we2_ref, be2_ref, out_ref):
    zp = zp_ref[...]
    xa = jnp.concatenate([zp[:, 0:A], zp[:, 2 * A:3 * A],
                          zp[:, 4 * A:5 * A]], axis=-1)
    xb = jnp.concatenate([zp[:, A:2 * A], zp[:, 3 * A:4 * A],
                          zp[:, 5 * A:6 * A]], axis=-1)
    x = jnp.concatenate([xa, xb], axis=0)
    ap = ang_ref[...]
    xang = jnp.concatenate([ap[:, :ANG], ap[:, ANG:]], axis=0)
    zz = jnp.dot(x, wfs_ref[...], preferred_element_type=jnp.float32)
    zz = zz + jnp.dot(xang, wang_ref[...], preferred_element_type=jnp.float32)
    zz = zz + bfs_ref[...]
    d = x[:, DCOL:DCOL + 1]
    expd = jnp.exp(d * d * (-1.0 / 18.0))
    g = jax.nn.sigmoid(zz[:, :A]) * jax.nn.softplus(zz[:, A:]) * expd
    cat = jnp.concatenate([g[:BE], g[BE:], ef_ref[...]], axis=-1)
    h = jnp.dot(cat, we1_ref[...], preferred_element_type=jnp.float32)
    h = jax.nn.silu(h + be1_ref[...])
    o = jnp.dot(h, we2_ref[...], preferred_element_type=jnp.float32)
    out_ref[...] = (o + be2_ref[...]).T


def _tc_compute(zp, ap, edge_fea, w_fs, w_ang, b_fs, w_e1, b_e1, w_e2, b_e2):
    nblk = N_EDGES // BE
    full = lambda shape: pl.BlockSpec(shape, lambda i: (0, 0))
    return pl.pallas_call(
        _tc_body,
        grid=(nblk,),
        in_specs=[
            pl.BlockSpec((BE, 2 * ZDIM), lambda i: (i, 0)),
            pl.BlockSpec((BE, 2 * ANG), lambda i: (i, 0)),
            pl.BlockSpec((BE, E_FEAT), lambda i: (i, 0)),
            full(w_fs.shape),
            full(w_ang.shape),
            full(b_fs.shape),
            full(w_e1.shape),
            full(b_e1.shape),
            full(w_e2.shape),
            full(b_e2.shape),
        ],
        out_specs=pl.BlockSpec((64, BE), lambda i: (0, i)),
        out_shape=jax.ShapeDtypeStruct((64, N_EDGES), jnp.float32),
        compiler_params=pltpu.CompilerParams(
            dimension_semantics=("parallel",),
        ),
    )(zp, ap, edge_fea, w_fs, w_ang, b_fs, w_e1, b_e1, w_e2, b_e2)


def kernel(atom_fea, edge_fea, sub_atom_idx, sub_edge_idx, sub_edge_ang,
           sub_index, distance, huge_structure, output_final_layer_neuron,
           W_f, b_f, W_s, b_s, W_e1, b_e1, W_e2, b_e2):
    sai = sub_atom_idx.astype(jnp.int32)
    ij = sub_edge_idx.astype(jnp.int32)
    i0 = sai[:, 0]
    i1 = sai[:, 1]
    edge_aug = jnp.concatenate(
        [edge_fea, distance[:, None],
         jnp.zeros((N_EDGES, A - E_FEAT - 1), jnp.float32)], axis=1)
    zp = _sc_assemble(atom_fea, edge_aug, i0, i1, ij)
    w_fs = jnp.concatenate([W_f, W_s], axis=1)
    w_fs_pad = w_fs.at[DCOL:, :].set(0.0)
    w_ang = w_fs[ZDIM - ANG:, :]
    b_fs = jnp.concatenate([b_f, b_s])[None, :]
    ap = sub_edge_ang.reshape(N_EDGES, 2 * ANG)
    out_t = _tc_compute(zp, ap, edge_fea, w_fs_pad, w_ang, b_fs,
                        W_e1, b_e1[None, :], W_e2, b_e2[None, :])
    return out_t.T


# speedup vs baseline: 1.0001x; 1.0001x over previous
import functools

import jax
import jax.numpy as jnp
from jax import lax
from jax.experimental import pallas as pl
from jax.experimental.pallas import tpu as pltpu
from jax.experimental.pallas import tpu_sc as plsc

N_NODES = 10000
N_EDGES = 160000
M = 2 * N_EDGES
A = 128
E_FEAT = 112
ANG = 16
ZDIM = 384

NC = 2
NS = 16
NW = NC * NS
CE = 64
NCHK = N_EDGES // CE
NT = (NCHK + NW - 1) // NW
NPAIR = (NT + 1) // 2


def _sc_assemble(atom_fea, edge_aug, i0, i1, ij):
    mesh = plsc.VectorSubcoreMesh(core_axis_name="c", subcore_axis_name="s")

    @functools.partial(
        pl.kernel,
        out_type=jax.ShapeDtypeStruct((N_EDGES, 6 * A), jnp.float32),
        mesh=mesh,
        scratch_types=[
            pltpu.VMEM((2 * CE,), jnp.int32),
            pltpu.VMEM((2 * CE,), jnp.int32),
            pltpu.VMEM((2 * CE,), jnp.int32),
            pltpu.VMEM((2 * CE,), jnp.int32),
            pltpu.VMEM((2 * CE,), jnp.int32),
            pltpu.VMEM((2 * CE,), jnp.int32),
            pltpu.VMEM((2 * CE, A), jnp.float32),
            pltpu.VMEM((2 * CE, A), jnp.float32),
            pltpu.VMEM((2 * CE, A), jnp.float32),
            pltpu.VMEM((2 * CE, A), jnp.float32),
            pltpu.VMEM((2 * CE, A), jnp.float32),
            pltpu.VMEM((2 * CE, A), jnp.float32),
            pltpu.SemaphoreType.DMA,
            pltpu.SemaphoreType.DMA,
            pltpu.SemaphoreType.DMA,
            pltpu.SemaphoreType.DMA,
            pltpu.SemaphoreType.DMA,
            pltpu.SemaphoreType.DMA,
        ],
    )
    def k(atom_hbm, edge_hbm, i0_hbm, i1_hbm, ij_hbm,
          z_hbm,
          x00, x01, x02, x10, x11, x12,
          b00, b01, b02, b10, b11, b12,
          sg0, sg1, sw0, sw1, si0, si1):
        wid = lax.axis_index("s") * NC + lax.axis_index("c")
        xraw = ((x00, x01, x02), (x10, x11, x12))
        bufs = ((b00, b01, b02), (b10, b11, b12))
        sg = (sg0, sg1)
        sw = (sw0, sw1)
        si = (si0, si1)
        idx_hbms = (i0_hbm, i1_hbm, ij_hbm)
        tabs = (atom_hbm, atom_hbm, edge_hbm)

        def chunk_of(t):
            return wid + t * NW

        def cond(t):
            return chunk_of(t) < NCHK

        def gathers_start(t, s):
            base = 2 * chunk_of(t) * CE
            cps = [pltpu.async_copy(idx_hbms[g].at[pl.ds(base, 2 * CE)],
                                    xraw[s][g], si[s]) for g in range(3)]
            for cp in cps:
                cp.wait()
            for g in range(3):
                pltpu.async_copy(tabs[g].at[xraw[s][g]], bufs[s][g], sg[s])

        def gathers_wait(s):
            for g in range(3):
                pltpu.make_async_copy(tabs[g].at[xraw[s][g]],
                                      bufs[s][g], sg[s]).wait()

        def writes_start(t, s):
            rows = pl.ds(chunk_of(t) * CE, CE)
            for g in range(3):
                pltpu.async_copy(bufs[s][g].reshape(CE, 2 * A),
                                 z_hbm.at[rows, pl.ds(g * 2 * A, 2 * A)],
                                 sw[s])

        def writes_wait(s):
            rows = pl.ds(0, CE)
            for g in range(3):
                pltpu.make_async_copy(bufs[s][g].reshape(CE, 2 * A),
                                      z_hbm.at[rows, pl.ds(g * 2 * A, 2 * A)],
                                      sw[s]).wait()

        gathers_start(0, 0)

        def pair(tt, carry):
            t0 = 2 * tt
            t1 = t0 + 1
            t2 = t0 + 2

            @pl.when(jnp.logical_and(cond(t1), t1 >= 3))
            def _():
                writes_wait(1)

            @pl.when(cond(t1))
            def _():
                gathers_start(t1, 1)

            @pl.when(cond(t0))
            def _():
                gathers_wait(0)
                writes_start(t0, 0)

            @pl.when(cond(t2))
            def _():
                writes_wait(0)
                gathers_start(t2, 0)

            @pl.when(cond(t1))
            def _():
                gathers_wait(1)
                writes_start(t1, 1)

            return carry

        lax.fori_loop(0, NPAIR, pair, 0)
        writes_wait(0)
        writes_wait(1)

    return k(atom_fea, edge_aug, i0, i1, ij)


BE = 3200
DCOL = 2 * A + E_FEAT


def _tc_body(zp_ref, ang_ref, ef_ref, wfs_ref, wang_ref, bfs_ref,
             we1_ref, be1_ref, we2_ref, be2_ref, out_ref):
    zp = zp_ref[...]
    xa = jnp.concatenate([zp[:, 0:A], zp[:, 2 * A:3 * A],
                          zp[:, 4 * A:5 * A]], axis=-1)
    xb = jnp.concatenate([zp[:, A:2 * A], zp[:, 3 * A:4 * A],
                          zp[:, 5 * A:6 * A]], axis=-1)
    x = jnp.concatenate([xa, xb], axis=0)
    ap = ang_ref[...]
    xang = jnp.concatenate([ap[:, :ANG], ap[:, ANG:]], axis=0)
    zz = jnp.dot(x, wfs_ref[...], preferred_element_type=jnp.float32)
    zz = zz + jnp.dot(xang, wang_ref[...], preferred_element_type=jnp.float32)
    zz = zz + bfs_ref[...]
    d = x[:, DCOL:DCOL + 1]
    expd = jnp.exp(d * d * (-1.0 / 18.0))
    g = jax.nn.sigmoid(zz[:, :A]) * jax.nn.softplus(zz[:, A:]) * expd
    cat = jnp.concatenate([g[:BE], g[BE:], ef_ref[...]], axis=-1)
    h = jnp.dot(cat, we1_ref[...], preferred_element_type=jnp.float32)
    h = jax.nn.silu(h + be1_ref[...])
    o = jnp.dot(h, we2_ref[...], preferred_element_type=jnp.float32)
    out_ref[...] = (o + be2_ref[...]).T


def _tc_compute(zp, ap, edge_fea, w_fs, w_ang, b_fs, w_e1, b_e1, w_e2, b_e2):
    nblk = N_EDGES // BE
    full = lambda shape: pl.BlockSpec(shape, lambda i: (0, 0))
    return pl.pallas_call(
        _tc_body,
        grid=(nblk,),
        in_specs=[
            pl.BlockSpec((BE, 2 * ZDIM), lambda i: (i, 0)),
            pl.BlockSpec((BE, 2 * ANG), lambda i: (i, 0)),
            pl.BlockSpec((BE, E_FEAT), lambda i: (i, 0)),
            full(w_fs.shape),
            full(w_ang.shape),
            full(b_fs.shape),
            full(w_e1.shape),
            full(b_e1.shape),
            full(w_e2.shape),
            full(b_e2.shape),
        ],
        out_specs=pl.BlockSpec((64, BE), lambda i: (0, i)),
        out_shape=jax.ShapeDtypeStruct((64, N_EDGES), jnp.float32),
        compiler_params=pltpu.CompilerParams(
            dimension_semantics=("parallel",),
        ),
    )(zp, ap, edge_fea, w_fs, w_ang, b_fs, w_e1, b_e1, w_e2, b_e2)


def kernel(atom_fea, edge_fea, sub_atom_idx, sub_edge_idx, sub_edge_ang,
           sub_index, distance, huge_structure, output_final_layer_neuron,
           W_f, b_f, W_s, b_s, W_e1, b_e1, W_e2, b_e2):
    sai = sub_atom_idx.astype(jnp.int32)
    ij = sub_edge_idx.astype(jnp.int32)
    i0 = sai[:, 0]
    i1 = sai[:, 1]
    edge_aug = jnp.concatenate(
        [edge_fea, distance[:, None],
         jnp.zeros((N_EDGES, A - E_FEAT - 1), jnp.float32)], axis=1)
    zp = _sc_assemble(atom_fea, edge_aug, i0, i1, ij)
    w_fs = jnp.concatenate([W_f, W_s], axis=1)
    w_fs_pad = w_fs.at[DCOL:, :].set(0.0)
    w_ang = w_fs[ZDIM - ANG:, :]
    b_fs = jnp.concatenate([b_f, b_s])[None, :]
    ap = sub_edge_ang.reshape(N_EDGES, 2 * ANG)
    out_t = _tc_compute(zp, ap, edge_fea, w_fs_pad, w_ang, b_fs,
                        W_e1, b_e1[None, :], W_e2, b_e2[None, :])
    return out_t.T
